# Initial kernel scaffold; baseline (speedup 1.0000x reference)
import functools

import jax
import jax.numpy as jnp
from jax import lax
from jax.experimental import pallas as pl
from jax.experimental.pallas import tpu as pltpu
from jax.experimental.pallas import tpu_sc as plsc

_D = 64
_NC = 2
_NS = 16
_NW = _NC * _NS
_C = 512


@functools.lru_cache(maxsize=None)
def _build(B: int):
    b_per_w = B // _NW
    n_chunks = b_per_w // _C
    mesh = plsc.VectorSubcoreMesh(
        core_axis_name="c", subcore_axis_name="s",
        num_cores=_NC, num_subcores=_NS)

    @functools.partial(
        pl.kernel,
        mesh=mesh,
        out_type=jax.ShapeDtypeStruct((B, _D), jnp.float32),
        scratch_types=[
            pltpu.VMEM((_C,), jnp.int32),
            pltpu.VMEM((_C, _D), jnp.float32),
            pltpu.SemaphoreType.DMA,
        ],
    )
    def emb(idx_hbm, table_hbm, out_hbm, idx_v, rows_v, sem):
        wid = lax.axis_index("s") * _NC + lax.axis_index("c")
        w_base = wid * b_per_w

        def body(i, carry):
            base = w_base + i * _C
            pltpu.sync_copy(idx_hbm.at[pl.ds(base, _C)], idx_v)
            pltpu.async_copy(table_hbm.at[idx_v], rows_v, sem).wait()
            pltpu.sync_copy(rows_v, out_hbm.at[pl.ds(base, _C)])
            return carry

        lax.fori_loop(0, n_chunks, body, 0)

    return emb


def kernel(x, table):
    orig_shape = x.shape
    flat = x.reshape(-1).astype(jnp.int32)
    out = _build(flat.shape[0])(flat, table)
    return out.reshape(*orig_shape, _D)


# speedup vs baseline: 1.7977x; 1.7977x over previous
import functools

import jax
import jax.numpy as jnp
from jax import lax
from jax.experimental import pallas as pl
from jax.experimental.pallas import tpu as pltpu
from jax.experimental.pallas import tpu_sc as plsc

_D = 64
_NC = 2
_NS = 16
_NW = _NC * _NS
_C = 512


@functools.lru_cache(maxsize=None)
def _build(B: int):
    b_per_w = B // _NW
    n_chunks = b_per_w // _C
    mesh = plsc.VectorSubcoreMesh(
        core_axis_name="c", subcore_axis_name="s",
        num_cores=_NC, num_subcores=_NS)

    @functools.partial(
        pl.kernel,
        mesh=mesh,
        out_type=jax.ShapeDtypeStruct((B, _D), jnp.float32),
        scratch_types=[
            pltpu.VMEM((_C,), jnp.int32),
            pltpu.VMEM((_C, _D), jnp.float32),
            pltpu.SemaphoreType.DMA,
        ],
        compiler_params=pltpu.CompilerParams(use_tc_tiling_on_sc=False),
    )
    def emb(idx_hbm, table_hbm, out_hbm, idx_v, rows_v, sem):
        wid = lax.axis_index("s") * _NC + lax.axis_index("c")
        w_base = wid * b_per_w

        def body(i, carry):
            base = w_base + i * _C
            pltpu.sync_copy(idx_hbm.at[pl.ds(base, _C)], idx_v)
            pltpu.async_copy(table_hbm.at[idx_v], rows_v, sem).wait()
            pltpu.sync_copy(rows_v, out_hbm.at[pl.ds(base, _C)])
            return carry

        lax.fori_loop(0, n_chunks, body, 0)

    return emb


def kernel(x, table):
    orig_shape = x.shape
    flat = x.reshape(-1).astype(jnp.int32)
    out = _build(flat.shape[0])(flat, table)
    return out.reshape(*orig_shape, _D)


# speedup vs baseline: 1.8746x; 1.0428x over previous
import functools

import jax
import jax.numpy as jnp
from jax import lax
from jax.experimental import pallas as pl
from jax.experimental.pallas import tpu as pltpu
from jax.experimental.pallas import tpu_sc as plsc

_D = 64
_NC = 2
_NS = 16
_NW = _NC * _NS
_C = 256
_NBUF = 4


@functools.lru_cache(maxsize=None)
def _build(B: int):
    b_per_w = B // _NW
    n_chunks = b_per_w // _C
    assert n_chunks % _NBUF == 0 and n_chunks >= 2 * _NBUF
    mesh = plsc.VectorSubcoreMesh(
        core_axis_name="c", subcore_axis_name="s",
        num_cores=_NC, num_subcores=_NS)

    @functools.partial(
        pl.kernel,
        mesh=mesh,
        out_type=jax.ShapeDtypeStruct((B, _D), jnp.float32),
        scratch_types=[
            pltpu.VMEM((_NBUF, _C), jnp.int32),
            pltpu.VMEM((_NBUF, _C, _D), jnp.float32),
            pltpu.SemaphoreType.DMA((_NBUF,)),
            pltpu.SemaphoreType.DMA((_NBUF,)),
            pltpu.SemaphoreType.DMA((_NBUF,)),
        ],
        compiler_params=pltpu.CompilerParams(use_tc_tiling_on_sc=False),
    )
    def emb(idx_hbm, table_hbm, out_hbm, idx_v, rows_v, sem_i, sem_g, sem_o):
        wid = lax.axis_index("s") * _NC + lax.axis_index("c")
        w_base = wid * b_per_w

        def idx_desc(g, b, make=False):
            f = pltpu.make_async_copy if make else pltpu.async_copy
            return f(idx_hbm.at[pl.ds(w_base + g * _C, _C)], idx_v.at[b],
                     sem_i.at[b])

        def gather_desc(b, make=False):
            f = pltpu.make_async_copy if make else pltpu.async_copy
            return f(table_hbm.at[idx_v.at[b]], rows_v.at[b], sem_g.at[b])

        def out_desc(g, b, make=False):
            f = pltpu.make_async_copy if make else pltpu.async_copy
            return f(rows_v.at[b], out_hbm.at[pl.ds(w_base + g * _C, _C)],
                     sem_o.at[b])

        for b in range(_NBUF - 1):
            idx_desc(b, b)

        def outer(i, carry):
            g0 = i * _NBUF
            for b in range(_NBUF):
                bp = (b - 1) % _NBUF
                g = g0 + b
                idx_desc(g, b, make=True).wait()
                @pl.when(g >= _NBUF)
                def _():
                    out_desc(g - _NBUF, b, make=True).wait()
                gather_desc(b)
                @pl.when(g >= 1)
                def _():
                    gather_desc(bp, make=True).wait()
                    out_desc(g - 1, bp)
                @pl.when(g + _NBUF - 1 < n_chunks)
                def _():
                    idx_desc(g + _NBUF - 1, bp)
            return carry

        lax.fori_loop(0, n_chunks // _NBUF, outer, 0)

        last = n_chunks - 1
        bl = last % _NBUF
        gather_desc(bl, make=True).wait()
        out_desc(last, bl)
        for g in range(n_chunks - _NBUF, n_chunks):
            out_desc(g, g % _NBUF, make=True).wait()

    return emb


def kernel(x, table):
    orig_shape = x.shape
    flat = x.reshape(-1).astype(jnp.int32)
    out = _build(flat.shape[0])(flat, table)
    return out.reshape(*orig_shape, _D)


# speedup vs baseline: 1.8759x; 1.0007x over previous
import functools

import jax
import jax.numpy as jnp
from jax import lax
from jax.experimental import pallas as pl
from jax.experimental.pallas import tpu as pltpu
from jax.experimental.pallas import tpu_sc as plsc

_D = 64
_NC = 2
_NS = 16
_NW = _NC * _NS
_C = 256
_NBUF = 5
_K = 3


@functools.lru_cache(maxsize=None)
def _build(B: int):
    b_per_w = B // _NW
    n_chunks = b_per_w // _C
    assert n_chunks * _C == b_per_w and n_chunks > _NBUF > _K
    mesh = plsc.VectorSubcoreMesh(
        core_axis_name="c", subcore_axis_name="s",
        num_cores=_NC, num_subcores=_NS)

    @functools.partial(
        pl.kernel,
        mesh=mesh,
        out_type=jax.ShapeDtypeStruct((B, _D), jnp.float32),
        scratch_types=[
            pltpu.VMEM((b_per_w,), jnp.int32),
            pltpu.VMEM((_NBUF, _C, _D), jnp.float32),
            pltpu.SemaphoreType.DMA,
            pltpu.SemaphoreType.DMA((_NBUF,)),
            pltpu.SemaphoreType.DMA((_NBUF,)),
        ],
        compiler_params=pltpu.CompilerParams(use_tc_tiling_on_sc=False),
    )
    def emb(idx_hbm, table_hbm, out_hbm, idx_v, rows_v, sem_i, sem_g, sem_o):
        wid = lax.axis_index("s") * _NC + lax.axis_index("c")
        w_base = wid * b_per_w

        def gather_desc(g, make=False):
            f = pltpu.make_async_copy if make else pltpu.async_copy
            b = lax.rem(g, _NBUF)
            return f(table_hbm.at[idx_v.at[pl.ds(g * _C, _C)]],
                     rows_v.at[b], sem_g.at[b])

        def out_desc(g, make=False):
            f = pltpu.make_async_copy if make else pltpu.async_copy
            b = lax.rem(g, _NBUF)
            return f(rows_v.at[b], out_hbm.at[pl.ds(w_base + g * _C, _C)],
                     sem_o.at[b])

        pltpu.async_copy(idx_hbm.at[pl.ds(w_base, b_per_w)], idx_v,
                         sem_i).wait()
        for g in range(_K):
            gather_desc(g)

        def body(g, carry):
            gather_desc(g, make=True).wait()
            out_desc(g)
            @pl.when(g + _K < n_chunks)
            def _():
                @pl.when(g + _K - _NBUF >= 0)
                def _():
                    out_desc(g + _K - _NBUF, make=True).wait()
                gather_desc(g + _K)
            return carry

        lax.fori_loop(0, n_chunks, body, 0)

        for g in range(n_chunks - _NBUF, n_chunks):
            out_desc(g, make=True).wait()

    return emb


def kernel(x, table):
    orig_shape = x.shape
    flat = x.reshape(-1).astype(jnp.int32)
    out = _build(flat.shape[0])(flat, table)
    return out.reshape(*orig_shape, _D)
